# Initial kernel scaffold; baseline (speedup 1.0000x reference)
import jax
import jax.numpy as jnp
from jax.experimental import pallas as pl


def kernel(x, edge_index, batch, edge_attr, W_head, b_head, W0, b0, W1, b1, W2, b2, Wo1, bo1, Wo2, bo2):
    raise NotImplementedError("write your pallas kernel here")


# speedup vs baseline: 2.6797x; 2.6797x over previous
import functools
import jax
import jax.numpy as jnp
from jax.experimental import pallas as pl
from jax.experimental.pallas import tpu as pltpu

N_NODES = 10000
NUM_SEG = 64


def _head_body(x_ref, w_ref, b_ref, dinv_ref, h_ref, g_ref):
    h = jnp.maximum(
        jnp.dot(x_ref[...], w_ref[...], preferred_element_type=jnp.float32)
        + b_ref[...], 0.0)
    h_ref[...] = h
    g_ref[...] = h * dinv_ref[...]


def _head(x, W, b, dinv, blk=1000):
    n = x.shape[0]
    grid = n // blk
    return pl.pallas_call(
        _head_body,
        grid=(grid,),
        in_specs=[
            pl.BlockSpec((blk, x.shape[1]), lambda i: (i, 0)),
            pl.BlockSpec(W.shape, lambda i: (0, 0)),
            pl.BlockSpec((1, b.shape[1]), lambda i: (0, 0)),
            pl.BlockSpec((blk, 1), lambda i: (i, 0)),
        ],
        out_specs=[
            pl.BlockSpec((blk, W.shape[1]), lambda i: (i, 0)),
            pl.BlockSpec((blk, W.shape[1]), lambda i: (i, 0)),
        ],
        out_shape=[
            jax.ShapeDtypeStruct((n, W.shape[1]), jnp.float32),
            jax.ShapeDtypeStruct((n, W.shape[1]), jnp.float32),
        ],
    )(x, W, b, dinv)


def _layer_body(agg_ref, g_ref, dinv_ref, w_ref, b_ref, h_ref, g2_ref):
    u = dinv_ref[...] * (agg_ref[...] + g_ref[...])
    m = jnp.dot(u, w_ref[...], preferred_element_type=jnp.float32) + b_ref[...]
    h = jnp.maximum(m, 0.0)
    h_ref[...] = h
    g2_ref[...] = h * dinv_ref[...]


def _layer(agg, g, dinv, W, b, blk=1000):
    n, d = agg.shape
    h = W.shape[1]
    grid = n // blk
    return pl.pallas_call(
        _layer_body,
        grid=(grid,),
        in_specs=[
            pl.BlockSpec((blk, d), lambda i: (i, 0)),
            pl.BlockSpec((blk, d), lambda i: (i, 0)),
            pl.BlockSpec((blk, 1), lambda i: (i, 0)),
            pl.BlockSpec((d, h), lambda i: (0, 0)),
            pl.BlockSpec((1, h), lambda i: (0, 0)),
        ],
        out_specs=[
            pl.BlockSpec((blk, h), lambda i: (i, 0)),
            pl.BlockSpec((blk, h), lambda i: (i, 0)),
        ],
        out_shape=[
            jax.ShapeDtypeStruct((n, h), jnp.float32),
            jax.ShapeDtypeStruct((n, h), jnp.float32),
        ],
    )(agg, g, dinv, W, b)


def _mlp_body(h1_ref, h2_ref, h3_ref, w1_ref, b1_ref, w2_ref, b2_ref, out_ref):
    outs = []
    for ref in (h1_ref, h2_ref, h3_ref):
        t = jnp.maximum(
            jnp.dot(ref[...], w1_ref[...], preferred_element_type=jnp.float32)
            + b1_ref[...], 0.0)
        o = jnp.dot(t, w2_ref[...], preferred_element_type=jnp.float32) + b2_ref[...]
        outs.append(o[:, None, :])
    out_ref[...] = jnp.concatenate(outs, axis=1)


def _mlp(h1, h2, h3, Wo1, bo1, Wo2, bo2, blk=1000):
    n, hdim = h1.shape
    odim = Wo2.shape[1]
    grid = n // blk
    return pl.pallas_call(
        _mlp_body,
        grid=(grid,),
        in_specs=[
            pl.BlockSpec((blk, hdim), lambda i: (i, 0)),
            pl.BlockSpec((blk, hdim), lambda i: (i, 0)),
            pl.BlockSpec((blk, hdim), lambda i: (i, 0)),
            pl.BlockSpec(Wo1.shape, lambda i: (0, 0)),
            pl.BlockSpec((1, bo1.shape[1]), lambda i: (0, 0)),
            pl.BlockSpec(Wo2.shape, lambda i: (0, 0)),
            pl.BlockSpec((1, bo2.shape[1]), lambda i: (0, 0)),
        ],
        out_specs=pl.BlockSpec((blk, 3, odim), lambda i: (i, 0, 0)),
        out_shape=jax.ShapeDtypeStruct((n, 3, odim), jnp.float32),
    )(h1, h2, h3, Wo1, bo1, Wo2, bo2)


def _pool_body(h_ref, batch_ref, out_ref, acc_ref, cnt_ref):
    i = pl.program_id(0)
    nprog = pl.num_programs(0)

    @pl.when(i == 0)
    def _():
        acc_ref[...] = jnp.zeros_like(acc_ref)
        cnt_ref[...] = jnp.zeros_like(cnt_ref)

    b = batch_ref[...]
    oh = (b == jax.lax.broadcasted_iota(jnp.int32, (1, NUM_SEG), 1)
          ).astype(jnp.float32)
    acc_ref[...] += jnp.dot(oh.T, h_ref[...], preferred_element_type=jnp.float32)
    cnt_ref[...] += jnp.sum(oh, axis=0)[:, None]

    @pl.when(i == nprog - 1)
    def _():
        out_ref[...] = acc_ref[...] / jnp.maximum(cnt_ref[...], 1.0)


def _pool(h, batch2d, blk=2000):
    n, hdim = h.shape
    grid = n // blk
    return pl.pallas_call(
        _pool_body,
        grid=(grid,),
        in_specs=[
            pl.BlockSpec((blk, hdim), lambda i: (i, 0)),
            pl.BlockSpec((blk, 1), lambda i: (i, 0)),
        ],
        out_specs=pl.BlockSpec((NUM_SEG, hdim), lambda i: (0, 0)),
        out_shape=jax.ShapeDtypeStruct((NUM_SEG, hdim), jnp.float32),
        scratch_shapes=[
            pltpu.VMEM((NUM_SEG, hdim), jnp.float32),
            pltpu.VMEM((NUM_SEG, 1), jnp.float32),
        ],
    )(h, batch2d)


def _aggregate(g, src, dst):
    return jnp.zeros((N_NODES, g.shape[1]), jnp.float32).at[dst].add(g[src])


def kernel(x, edge_index, batch, edge_attr, W_head, b_head, W0, b0, W1, b1,
           W2, b2, Wo1, bo1, Wo2, bo2):
    src = edge_index[0]
    dst = edge_index[1]

    deg = jnp.zeros((N_NODES,), jnp.float32).at[dst].add(1.0) + 1.0
    dinv = jax.lax.rsqrt(deg)[:, None]

    b_head2 = b_head[None, :]
    h0, g0 = _head(x, W_head, b_head2, dinv)

    agg0 = _aggregate(g0, src, dst)
    h1, g1 = _layer(agg0, g0, dinv, W0, b0[None, :])
    agg1 = _aggregate(g1, src, dst)
    h2, g2 = _layer(agg1, g1, dinv, W1, b1[None, :])
    agg2 = _aggregate(g2, src, dst)
    h3, _ = _layer(agg2, g2, dinv, W2, b2[None, :])

    emb_n = _mlp(h1, h2, h3, Wo1, bo1[None, :], Wo2, bo2[None, :])
    emb_g = _pool(h3, batch[:, None].astype(jnp.int32))

    return (emb_g[:, None, :], emb_n, None)


# speedup vs baseline: 4.5940x; 1.7144x over previous
import functools
import jax
import jax.numpy as jnp
from jax import lax
from jax.experimental import pallas as pl
from jax.experimental.pallas import tpu as pltpu
from jax.experimental.pallas import tpu_sc as plsc

N_NODES = 10000
N_PAD = 10240
NUM_SEG = 64
E_TOT = 320000
EB = 80
ER = 4096
E_PAD = ER * EB
N_SINK = 10496
PART = 2560
QTHRESH = 3968
QCAP = 4160

_MESH = plsc.VectorSubcoreMesh(core_axis_name="c", subcore_axis_name="s")
_SC_PARAMS = pltpu.CompilerParams(needs_layout_passes=False)
_NC = 2
_NS = 16


def _zero_vmem(ref, rows, cols):
    z = jnp.zeros((16,), jnp.float32)

    def body(i, _):
        r = i // (cols // 16)
        cidx = (i % (cols // 16)) * 16
        ref[r, pl.ds(cidx, 16)] = z
        return 0

    lax.fori_loop(0, rows * (cols // 16), body, 0)


def _zero_vmem3(ref):
    z = jnp.zeros((16,), jnp.float32)

    def body(i, _):
        r = i // 32
        q = (i % 32) // 8
        cidx = (i % 8) * 16
        ref[r, q, pl.ds(cidx, 16)] = z
        return 0

    lax.fori_loop(0, 16 * 4 * 8, body, 0)


def _sc_deg_body(dst_hbm, out_hbm, dst_v, ones_v, zb_v, deg_sh, deg_out_v):
    c = lax.axis_index("c")
    s = lax.axis_index("s")
    wid = s * _NC + c

    def zb(i, _):
        zb_v[pl.ds(i * 16, 16)] = jnp.zeros((16,), jnp.float32)
        return 0
    lax.fori_loop(0, 41, zb, 0)

    def ob(i, _):
        ones_v[pl.ds(i * 16, 16)] = jnp.ones((16,), jnp.float32)
        return 0
    lax.fori_loop(0, EB // 16, ob, 0)

    pltpu.sync_copy(zb_v, deg_sh.at[pl.ds(s * 656, 656)])
    plsc.subcore_barrier()

    rows = ER // (_NC * _NS)
    pltpu.sync_copy(dst_hbm.at[pl.ds(wid * rows, rows)], dst_v)

    def body(j, _):
        pltpu.sync_copy(ones_v, deg_sh.at[dst_v.at[j]], add=True)
        return 0
    lax.fori_loop(0, rows, body, 0)

    plsc.subcore_barrier()
    pltpu.sync_copy(deg_sh.at[pl.ds(s * 640, 640)], deg_out_v)
    pltpu.sync_copy(deg_out_v, out_hbm.at[c, pl.ds(s * 640, 640)])


def _sc_deg(dst2d):
    f = functools.partial(
        pl.kernel,
        out_type=jax.ShapeDtypeStruct((2, N_PAD), jnp.float32),
        compiler_params=_SC_PARAMS,
        mesh=_MESH,
        scratch_types=[
            pltpu.VMEM((ER // 32, EB), jnp.int32),
            pltpu.VMEM((EB,), jnp.float32),
            pltpu.VMEM((656,), jnp.float32),
            pltpu.VMEM_SHARED((N_SINK,), jnp.float32),
            pltpu.VMEM((640,), jnp.float32),
        ],
    )(_sc_deg_body)
    return f(dst2d)


def _sc_agg128_body(src_hbm, dst_hbm, g_hbm, out_hbm,
                    src_v, dst_v, bufs, agg_sh, out_v, gsems, ssems):
    c = lax.axis_index("c")
    s = lax.axis_index("s")
    wid = s * _NC + c

    _zero_vmem(out_v, 16, 128)
    for k in range(41):
        pltpu.sync_copy(out_v, agg_sh.at[pl.ds(s * 656 + k * 16, 16), :])
    plsc.subcore_barrier()

    rows = ER // (_NC * _NS)
    pltpu.sync_copy(src_hbm.at[pl.ds(wid * rows, rows)], src_v)
    pltpu.sync_copy(dst_hbm.at[pl.ds(wid * rows, rows)], dst_v)

    def body(j, _):
        gh = []
        for k in range(5):
            si = src_v[j, pl.ds(k * 16, 16)]
            gh.append(pltpu.async_copy(g_hbm.at[si], bufs[k], gsems[k]))
        sh = []
        for k in range(5):
            di = dst_v[j, pl.ds(k * 16, 16)]
            gh[k].wait()
            sh.append(pltpu.async_copy(bufs[k], agg_sh.at[di], ssems[k],
                                       add=True))
        for k in range(5):
            sh[k].wait()
        return 0
    lax.fori_loop(0, rows, body, 0)

    plsc.subcore_barrier()

    def obody(k, _):
        pltpu.sync_copy(agg_sh.at[pl.ds(s * 640 + k * 16, 16), :], out_v)
        pltpu.sync_copy(out_v, out_hbm.at[c, pl.ds(s * 640 + k * 16, 16), :])
        return 0
    lax.fori_loop(0, 40, obody, 0)


def _sc_agg128(src2d, dst2d, g):
    f = functools.partial(
        pl.kernel,
        out_type=jax.ShapeDtypeStruct((2, N_PAD, 128), jnp.float32),
        compiler_params=_SC_PARAMS,
        mesh=_MESH,
        scratch_types=[
            pltpu.VMEM((ER // 32, EB), jnp.int32),
            pltpu.VMEM((ER // 32, EB), jnp.int32),
            [pltpu.VMEM((16, 128), jnp.float32) for _ in range(5)],
            pltpu.VMEM_SHARED((N_SINK, 128), jnp.float32),
            pltpu.VMEM((16, 128), jnp.float32),
            [pltpu.SemaphoreType.DMA for _ in range(5)],
            [pltpu.SemaphoreType.DMA for _ in range(5)],
        ],
    )(_sc_agg128_body)
    return f(src2d, dst2d, g)


def _sc_agg512_body(src_hbm, dst_hbm, g_hbm, out_hbm,
                    sbuf, dbuf, q_v, sidx_v, didx_v, bufs, part_sh,
                    gsems, ssems):
    c = lax.axis_index("c")
    s = lax.axis_index("s")

    rows = ER // _NS
    for b in bufs:
        _zero_vmem3(b)
    mask14 = jnp.full((16,), 16383, jnp.int32)
    partu = jnp.full((16,), PART, jnp.uint32)
    maxi = jnp.full((16,), 0x7FFFFFF, jnp.int32)

    def drain(off):
        sinkp = jnp.full((16,), (PART + s) << 14, jnp.int32)
        for t in range(4):
            q_v[pl.ds(off + t * 16, 16)] = sinkp
        nb = (off + 63) // 64

        def gb(t, _):
            base = t * 64
            for k in range(4):
                qv = q_v[pl.ds(base + k * 16, 16)]
                sidx_v[k, :] = jnp.bitwise_and(qv, mask14)
                didx_v[k, :] = jnp.right_shift(qv, 14)
            gh = []
            for k in range(4):
                si = sidx_v[k, :]
                gh.append(pltpu.async_copy(g_hbm.at[si], bufs[k], gsems[k]))
            sh = []
            for k in range(4):
                di = didx_v[k, :]
                gh[k].wait()
                sh.append(pltpu.async_copy(bufs[k], part_sh.at[di], ssems[k],
                                           add=True))
            for k in range(4):
                sh[k].wait()
            return 0
        lax.fori_loop(0, nb, gb, 0)

    for p in range(2):
        pg = c * 2 + p
        lo = pg * PART

        for k in range(10):
            pltpu.sync_copy(bufs[0],
                            part_sh.at[pl.ds(s * 160 + k * 16, 16)])
        plsc.subcore_barrier()

        lo_v = jnp.full((16,), lo, jnp.int32)

        def fchunk(t, _):
            pltpu.sync_copy(src_hbm.at[pl.ds(s * rows + t * 16, 16)], sbuf)
            pltpu.sync_copy(dst_hbm.at[pl.ds(s * rows + t * 16, 16)], dbuf)
            sinkrel = jnp.full((16,), PART + s, jnp.int32)
            for r in range(16):
                for k in range(5):
                    dv = dbuf[r, pl.ds(k * 16, 16)]
                    sv = sbuf[r, pl.ds(k * 16, 16)]
                    rel = dv - lo_v
                    m = plsc.bitcast(rel, jnp.uint32) < partu
                    relx = jnp.where(m, rel, sinkrel)
                    packed = jnp.bitwise_or(jnp.left_shift(relx, 14), sv)
                    q_v[pl.ds((r * 5 + k) * 16, 16)] = packed
            def gb(t2, _):
                base = t2 * 64
                for k in range(4):
                    qv = q_v[pl.ds(base + k * 16, 16)]
                    sidx_v[k, :] = jnp.bitwise_and(qv, mask14)
                    didx_v[k, :] = jnp.right_shift(qv, 14)
                gh = []
                for k in range(4):
                    si = sidx_v[k, :]
                    gh.append(pltpu.async_copy(g_hbm.at[si], bufs[k], gsems[k]))
                sh = []
                for k in range(4):
                    di = didx_v[k, :]
                    gh[k].wait()
                    sh.append(pltpu.async_copy(bufs[k], part_sh.at[di],
                                               ssems[k], add=True))
                for k in range(4):
                    sh[k].wait()
                return 0
            lax.fori_loop(0, 20, gb, 0)
            return 0
        lax.fori_loop(0, rows // 16, fchunk, 0)

        plsc.subcore_barrier()
        def obody(k, _):
            r = s * 160 + k * 16
            pltpu.sync_copy(part_sh.at[pl.ds(r, 16)], bufs[1])
            pltpu.sync_copy(bufs[1], out_hbm.at[pl.ds(lo + r, 16)])
            return 0
        lax.fori_loop(0, 10, obody, 0)
        _zero_vmem3(bufs[0])
        _zero_vmem3(bufs[1])
        plsc.subcore_barrier()


def _sc_agg512(src2d, dst2d, g):
    f = functools.partial(
        pl.kernel,
        out_type=jax.ShapeDtypeStruct((N_PAD, 4, 128), jnp.float32),
        compiler_params=_SC_PARAMS,
        mesh=_MESH,
        scratch_types=[
            pltpu.VMEM((16, EB), jnp.int32),
            pltpu.VMEM((16, EB), jnp.int32),
            pltpu.VMEM((QCAP,), jnp.int32),
            pltpu.VMEM((4, 16), jnp.int32),
            pltpu.VMEM((4, 16), jnp.int32),
            [pltpu.VMEM((16, 4, 128), jnp.float32) for _ in range(4)],
            pltpu.VMEM_SHARED((PART + 16, 4, 128), jnp.float32),
            [pltpu.SemaphoreType.DMA for _ in range(4)],
            [pltpu.SemaphoreType.DMA for _ in range(4)],
        ],
    )(_sc_agg512_body)
    return f(src2d, dst2d, g.reshape(N_PAD, 4, 128)).reshape(N_PAD, 512)


def _dinv_body(deg_ref, out_ref):
    d = deg_ref[:, 0:1] + deg_ref[:, 1:2] + 1.0
    out_ref[...] = jax.lax.rsqrt(d)


def _dinv(deg2t):
    return pl.pallas_call(
        _dinv_body,
        in_specs=[pl.BlockSpec((N_PAD, 2), lambda: (0, 0))],
        out_specs=pl.BlockSpec((N_PAD, 1), lambda: (0, 0)),
        out_shape=jax.ShapeDtypeStruct((N_PAD, 1), jnp.float32),
    )(deg2t)


def _head_body(x_ref, w_ref, b_ref, dinv_ref, h_ref, g_ref):
    h = jnp.maximum(
        jnp.dot(x_ref[...], w_ref[...], preferred_element_type=jnp.float32)
        + b_ref[...], 0.0)
    h_ref[...] = h
    g_ref[...] = h * dinv_ref[...]


def _head(x, W, b, dinv, blk=1280):
    n = x.shape[0]
    return pl.pallas_call(
        _head_body,
        grid=(n // blk,),
        in_specs=[
            pl.BlockSpec((blk, x.shape[1]), lambda i: (i, 0)),
            pl.BlockSpec(W.shape, lambda i: (0, 0)),
            pl.BlockSpec((1, b.shape[1]), lambda i: (0, 0)),
            pl.BlockSpec((blk, 1), lambda i: (i, 0)),
        ],
        out_specs=[
            pl.BlockSpec((blk, W.shape[1]), lambda i: (i, 0)),
            pl.BlockSpec((blk, W.shape[1]), lambda i: (i, 0)),
        ],
        out_shape=[
            jax.ShapeDtypeStruct((n, W.shape[1]), jnp.float32),
            jax.ShapeDtypeStruct((n, W.shape[1]), jnp.float32),
        ],
    )(x, W, b, dinv)


def _layer0_body(agg_ref, g_ref, dinv_ref, w_ref, b_ref, h_ref, g2_ref):
    u = dinv_ref[...] * (agg_ref[0] + agg_ref[1] + g_ref[...])
    m = jnp.dot(u, w_ref[...], preferred_element_type=jnp.float32) + b_ref[...]
    h = jnp.maximum(m, 0.0)
    h_ref[...] = h
    g2_ref[...] = h * dinv_ref[...]


def _layer0(agg, g, dinv, W, b, blk=1280):
    n, d = g.shape
    h = W.shape[1]
    return pl.pallas_call(
        _layer0_body,
        grid=(n // blk,),
        in_specs=[
            pl.BlockSpec((2, blk, d), lambda i: (0, i, 0)),
            pl.BlockSpec((blk, d), lambda i: (i, 0)),
            pl.BlockSpec((blk, 1), lambda i: (i, 0)),
            pl.BlockSpec((d, h), lambda i: (0, 0)),
            pl.BlockSpec((1, h), lambda i: (0, 0)),
        ],
        out_specs=[
            pl.BlockSpec((blk, h), lambda i: (i, 0)),
            pl.BlockSpec((blk, h), lambda i: (i, 0)),
        ],
        out_shape=[
            jax.ShapeDtypeStruct((n, h), jnp.float32),
            jax.ShapeDtypeStruct((n, h), jnp.float32),
        ],
    )(agg, g, dinv, W, b)


def _layer_body(agg_ref, g_ref, dinv_ref, w_ref, b_ref, h_ref, g2_ref):
    u = dinv_ref[...] * (agg_ref[...] + g_ref[...])
    m = jnp.dot(u, w_ref[...], preferred_element_type=jnp.float32) + b_ref[...]
    h = jnp.maximum(m, 0.0)
    h_ref[...] = h
    g2_ref[...] = h * dinv_ref[...]


def _layer(agg, g, dinv, W, b, blk=1280):
    n, d = agg.shape
    h = W.shape[1]
    return pl.pallas_call(
        _layer_body,
        grid=(n // blk,),
        in_specs=[
            pl.BlockSpec((blk, d), lambda i: (i, 0)),
            pl.BlockSpec((blk, d), lambda i: (i, 0)),
            pl.BlockSpec((blk, 1), lambda i: (i, 0)),
            pl.BlockSpec((d, h), lambda i: (0, 0)),
            pl.BlockSpec((1, h), lambda i: (0, 0)),
        ],
        out_specs=[
            pl.BlockSpec((blk, h), lambda i: (i, 0)),
            pl.BlockSpec((blk, h), lambda i: (i, 0)),
        ],
        out_shape=[
            jax.ShapeDtypeStruct((n, h), jnp.float32),
            jax.ShapeDtypeStruct((n, h), jnp.float32),
        ],
    )(agg, g, dinv, W, b)


def _mlp_body(h1_ref, h2_ref, h3_ref, w1_ref, b1_ref, w2_ref, b2_ref, out_ref):
    outs = []
    for ref in (h1_ref, h2_ref, h3_ref):
        t = jnp.maximum(
            jnp.dot(ref[...], w1_ref[...], preferred_element_type=jnp.float32)
            + b1_ref[...], 0.0)
        o = jnp.dot(t, w2_ref[...], preferred_element_type=jnp.float32) + b2_ref[...]
        outs.append(o[:, None, :])
    out_ref[...] = jnp.concatenate(outs, axis=1)


def _mlp(h1, h2, h3, Wo1, bo1, Wo2, bo2, blk=400):
    n, hdim = h1.shape
    odim = Wo2.shape[1]
    return pl.pallas_call(
        _mlp_body,
        grid=(N_NODES // blk,),
        in_specs=[
            pl.BlockSpec((blk, hdim), lambda i: (i, 0)),
            pl.BlockSpec((blk, hdim), lambda i: (i, 0)),
            pl.BlockSpec((blk, hdim), lambda i: (i, 0)),
            pl.BlockSpec(Wo1.shape, lambda i: (0, 0)),
            pl.BlockSpec((1, bo1.shape[1]), lambda i: (0, 0)),
            pl.BlockSpec(Wo2.shape, lambda i: (0, 0)),
            pl.BlockSpec((1, bo2.shape[1]), lambda i: (0, 0)),
        ],
        out_specs=pl.BlockSpec((blk, 3, odim), lambda i: (i, 0, 0)),
        out_shape=jax.ShapeDtypeStruct((N_NODES, 3, odim), jnp.float32),
    )(h1, h2, h3, Wo1, bo1, Wo2, bo2)


def _pool_body(h_ref, batch_ref, out_ref, acc_ref, cnt_ref):
    i = pl.program_id(0)
    nprog = pl.num_programs(0)

    @pl.when(i == 0)
    def _():
        acc_ref[...] = jnp.zeros_like(acc_ref)
        cnt_ref[...] = jnp.zeros_like(cnt_ref)

    b = batch_ref[...]
    oh = (b == jax.lax.broadcasted_iota(jnp.int32, (1, NUM_SEG), 1)
          ).astype(jnp.float32)
    acc_ref[...] += jnp.dot(oh.T, h_ref[...], preferred_element_type=jnp.float32)
    cnt_ref[...] += jnp.sum(oh, axis=0)[:, None]

    @pl.when(i == nprog - 1)
    def _():
        out_ref[...] = acc_ref[...] / jnp.maximum(cnt_ref[...], 1.0)


def _pool(h, batch2d, blk=2000):
    hdim = h.shape[1]
    return pl.pallas_call(
        _pool_body,
        grid=(N_NODES // blk,),
        in_specs=[
            pl.BlockSpec((blk, hdim), lambda i: (i, 0)),
            pl.BlockSpec((blk, 1), lambda i: (i, 0)),
        ],
        out_specs=pl.BlockSpec((NUM_SEG, hdim), lambda i: (0, 0)),
        out_shape=jax.ShapeDtypeStruct((NUM_SEG, hdim), jnp.float32),
        scratch_shapes=[
            pltpu.VMEM((NUM_SEG, hdim), jnp.float32),
            pltpu.VMEM((NUM_SEG, 1), jnp.float32),
        ],
    )(h, batch2d)


def kernel(x, edge_index, batch, edge_attr, W_head, b_head, W0, b0, W1, b1,
           W2, b2, Wo1, bo1, Wo2, bo2):
    npad_e = E_PAD - E_TOT
    pad_src = jnp.arange(npad_e, dtype=jnp.int32) % N_NODES
    pad_dst = N_PAD + jnp.arange(npad_e, dtype=jnp.int32) % 256
    src2d = jnp.concatenate(
        [edge_index[0].astype(jnp.int32), pad_src]).reshape(ER, EB)
    dst2d = jnp.concatenate(
        [edge_index[1].astype(jnp.int32), pad_dst]).reshape(ER, EB)
    xp = jnp.concatenate(
        [x, jnp.zeros((N_PAD - N_NODES, x.shape[1]), jnp.float32)], axis=0)
    batchp = batch[:, None].astype(jnp.int32)

    deg2 = _sc_deg(dst2d)
    dinv = _dinv(deg2.T)

    h0, g0 = _head(xp, W_head, b_head[None, :], dinv)
    agg0 = _sc_agg128(src2d, dst2d, g0)
    h1, g1 = _layer0(agg0, g0, dinv, W0, b0[None, :])
    agg1 = _sc_agg512(src2d, dst2d, g1)
    h2, g2 = _layer(agg1, g1, dinv, W1, b1[None, :])
    agg2 = _sc_agg512(src2d, dst2d, g2)
    h3, _ = _layer(agg2, g2, dinv, W2, b2[None, :])

    emb_n = _mlp(h1, h2, h3, Wo1, bo1[None, :], Wo2, bo2[None, :])
    emb_g = _pool(h3, batchp)

    return (emb_g[:, None, :], emb_n, None)


# speedup vs baseline: 6.9799x; 1.5193x over previous
import functools
import jax
import jax.numpy as jnp
from jax import lax
from jax.experimental import pallas as pl
from jax.experimental.pallas import tpu as pltpu
from jax.experimental.pallas import tpu_sc as plsc

N_NODES = 10000
N_PAD = 10240
NUM_SEG = 64
E_TOT = 320000
EB = 80
ER = 4096
E_PAD = ER * EB
N_SINK = 10496
PART = 2560
HPART = 5120
QTHRESH = 3968
QCAP = 4160

_MESH = plsc.VectorSubcoreMesh(core_axis_name="c", subcore_axis_name="s")
_SC_PARAMS = pltpu.CompilerParams(needs_layout_passes=False)
_NC = 2
_NS = 16


def _zero_vmem(ref, rows, cols):
    z = jnp.zeros((16,), jnp.float32)

    def body(i, _):
        r = i // (cols // 16)
        cidx = (i % (cols // 16)) * 16
        ref[r, pl.ds(cidx, 16)] = z
        return 0

    lax.fori_loop(0, rows * (cols // 16), body, 0)


def _zero_vmem3(ref):
    z = jnp.zeros((16,), jnp.float32)

    def body(i, _):
        r = i // 32
        q = (i % 32) // 8
        cidx = (i % 8) * 16
        ref[r, q, pl.ds(cidx, 16)] = z
        return 0

    lax.fori_loop(0, 16 * 4 * 8, body, 0)


def _sc_deg_body(dst_hbm, out_hbm, dst_v, ones_v, zb_v, deg_sh, deg_out_v):
    c = lax.axis_index("c")
    s = lax.axis_index("s")
    wid = s * _NC + c

    def zb(i, _):
        zb_v[pl.ds(i * 16, 16)] = jnp.zeros((16,), jnp.float32)
        return 0
    lax.fori_loop(0, 41, zb, 0)

    def ob(i, _):
        ones_v[pl.ds(i * 16, 16)] = jnp.ones((16,), jnp.float32)
        return 0
    lax.fori_loop(0, EB // 16, ob, 0)

    pltpu.sync_copy(zb_v, deg_sh.at[pl.ds(s * 656, 656)])
    plsc.subcore_barrier()

    rows = ER // (_NC * _NS)
    pltpu.sync_copy(dst_hbm.at[pl.ds(wid * rows, rows)], dst_v)

    def body(j, _):
        pltpu.sync_copy(ones_v, deg_sh.at[dst_v.at[j]], add=True)
        return 0
    lax.fori_loop(0, rows, body, 0)

    plsc.subcore_barrier()
    pltpu.sync_copy(deg_sh.at[pl.ds(s * 640, 640)], deg_out_v)
    pltpu.sync_copy(deg_out_v, out_hbm.at[c, pl.ds(s * 640, 640)])


def _sc_deg(dst2d):
    f = functools.partial(
        pl.kernel,
        out_type=jax.ShapeDtypeStruct((2, N_PAD), jnp.float32),
        compiler_params=_SC_PARAMS,
        mesh=_MESH,
        scratch_types=[
            pltpu.VMEM((ER // 32, EB), jnp.int32),
            pltpu.VMEM((EB,), jnp.float32),
            pltpu.VMEM((656,), jnp.float32),
            pltpu.VMEM_SHARED((N_SINK,), jnp.float32),
            pltpu.VMEM((640,), jnp.float32),
        ],
    )(_sc_deg_body)
    return f(dst2d)


def _sc_agg128_body(src_hbm, dst_hbm, g_hbm, out_hbm,
                    src_v, dst_v, bufs, agg_sh, out_v, gsems, ssems):
    c = lax.axis_index("c")
    s = lax.axis_index("s")
    wid = s * _NC + c

    _zero_vmem(out_v, 16, 128)
    for k in range(41):
        pltpu.sync_copy(out_v, agg_sh.at[pl.ds(s * 656 + k * 16, 16), :])
    plsc.subcore_barrier()

    rows = ER // (_NC * _NS)
    pltpu.sync_copy(src_hbm.at[pl.ds(wid * rows, rows)], src_v)
    pltpu.sync_copy(dst_hbm.at[pl.ds(wid * rows, rows)], dst_v)

    def body(j, _):
        gh = []
        for k in range(5):
            si = src_v[j, pl.ds(k * 16, 16)]
            gh.append(pltpu.async_copy(g_hbm.at[si], bufs[k], gsems[k]))
        sh = []
        for k in range(5):
            di = dst_v[j, pl.ds(k * 16, 16)]
            gh[k].wait()
            sh.append(pltpu.async_copy(bufs[k], agg_sh.at[di], ssems[k],
                                       add=True))
        for k in range(5):
            sh[k].wait()
        return 0
    lax.fori_loop(0, rows, body, 0)

    plsc.subcore_barrier()

    def obody(k, _):
        pltpu.sync_copy(agg_sh.at[pl.ds(s * 640 + k * 16, 16), :], out_v)
        pltpu.sync_copy(out_v, out_hbm.at[c, pl.ds(s * 640 + k * 16, 16), :])
        return 0
    lax.fori_loop(0, 40, obody, 0)


def _sc_agg128(src2d, dst2d, g):
    f = functools.partial(
        pl.kernel,
        out_type=jax.ShapeDtypeStruct((2, N_PAD, 128), jnp.float32),
        compiler_params=_SC_PARAMS,
        mesh=_MESH,
        scratch_types=[
            pltpu.VMEM((ER // 32, EB), jnp.int32),
            pltpu.VMEM((ER // 32, EB), jnp.int32),
            [pltpu.VMEM((16, 128), jnp.float32) for _ in range(5)],
            pltpu.VMEM_SHARED((N_SINK, 128), jnp.float32),
            pltpu.VMEM((16, 128), jnp.float32),
            [pltpu.SemaphoreType.DMA for _ in range(5)],
            [pltpu.SemaphoreType.DMA for _ in range(5)],
        ],
    )(_sc_agg128_body)
    return f(src2d, dst2d, g)


def _zero_vmem3h(ref):
    z = jnp.zeros((16,), jnp.float32)

    def body(i, _):
        r = i // 16
        q = (i % 16) // 8
        cidx = (i % 8) * 16
        ref[r, q, pl.ds(cidx, 16)] = z
        return 0

    lax.fori_loop(0, 16 * 2 * 8, body, 0)


def _sc_agg512_body(src_hbm, dst_hbm, ga_hbm, gb_hbm, outa_hbm, outb_hbm,
                    sbuf, dbuf, q_v, sidx_v, didx_v, bufs, part_sh,
                    gsems, ssems):
    c = lax.axis_index("c")
    s = lax.axis_index("s")

    rows = ER // _NS
    for b in bufs:
        _zero_vmem3h(b)
    mask14 = jnp.full((16,), 16383, jnp.int32)
    halfu = jnp.full((16,), HPART, jnp.uint32)
    lo_v = jnp.full((16,), c * HPART, jnp.int32)

    for h, (g_hbm, out_hbm) in enumerate(((ga_hbm, outa_hbm),
                                          (gb_hbm, outb_hbm))):
        def zbody(k, _):
            pltpu.sync_copy(bufs[0],
                            part_sh.at[pl.ds(s * 320 + k * 16, 16)])
            return 0
        lax.fori_loop(0, 20, zbody, 0)
        plsc.subcore_barrier()

        def fchunk(t, _):
            pltpu.sync_copy(src_hbm.at[pl.ds(s * rows + t * 16, 16)], sbuf)
            pltpu.sync_copy(dst_hbm.at[pl.ds(s * rows + t * 16, 16)], dbuf)
            sinkrel = jnp.full((16,), HPART + s, jnp.int32)
            for r in range(16):
                for k in range(5):
                    dv = dbuf[r, pl.ds(k * 16, 16)]
                    sv = sbuf[r, pl.ds(k * 16, 16)]
                    rel = dv - lo_v
                    m = plsc.bitcast(rel, jnp.uint32) < halfu
                    relx = jnp.where(m, rel, sinkrel)
                    packed = jnp.bitwise_or(jnp.left_shift(relx, 14), sv)
                    q_v[pl.ds((r * 5 + k) * 16, 16)] = packed

            def gb2(t2, _):
                base = t2 * 64
                for k in range(4):
                    qv = q_v[pl.ds(base + k * 16, 16)]
                    sidx_v[k, :] = jnp.bitwise_and(qv, mask14)
                    didx_v[k, :] = jnp.right_shift(qv, 14)
                gh = []
                for k in range(4):
                    si = sidx_v[k, :]
                    gh.append(pltpu.async_copy(g_hbm.at[si], bufs[k],
                                               gsems[k]))
                sh = []
                for k in range(4):
                    di = didx_v[k, :]
                    gh[k].wait()
                    sh.append(pltpu.async_copy(bufs[k], part_sh.at[di],
                                               ssems[k], add=True))
                for k in range(4):
                    sh[k].wait()
                return 0
            lax.fori_loop(0, 20, gb2, 0)
            return 0
        lax.fori_loop(0, rows // 16, fchunk, 0)

        plsc.subcore_barrier()
        def obody(k, _):
            r = s * 320 + k * 16
            pltpu.sync_copy(part_sh.at[pl.ds(r, 16)], bufs[1])
            pltpu.sync_copy(bufs[1], out_hbm.at[pl.ds(c * HPART + r, 16)])
            return 0
        lax.fori_loop(0, 20, obody, 0)
        _zero_vmem3h(bufs[0])
        _zero_vmem3h(bufs[1])
        plsc.subcore_barrier()


def _sc_agg512(src2d, dst2d, g):
    ga = g[:, :256].reshape(N_PAD, 2, 128)
    gb = g[:, 256:].reshape(N_PAD, 2, 128)
    f = functools.partial(
        pl.kernel,
        out_type=[jax.ShapeDtypeStruct((N_PAD, 2, 128), jnp.float32),
                  jax.ShapeDtypeStruct((N_PAD, 2, 128), jnp.float32)],
        compiler_params=_SC_PARAMS,
        mesh=_MESH,
        scratch_types=[
            pltpu.VMEM((16, EB), jnp.int32),
            pltpu.VMEM((16, EB), jnp.int32),
            pltpu.VMEM((1280,), jnp.int32),
            pltpu.VMEM((4, 16), jnp.int32),
            pltpu.VMEM((4, 16), jnp.int32),
            [pltpu.VMEM((16, 2, 128), jnp.float32) for _ in range(4)],
            pltpu.VMEM_SHARED((HPART + 16, 2, 128), jnp.float32),
            [pltpu.SemaphoreType.DMA for _ in range(4)],
            [pltpu.SemaphoreType.DMA for _ in range(4)],
        ],
    )(_sc_agg512_body)
    oa, ob = f(src2d, dst2d, ga, gb)
    return jnp.concatenate([oa.reshape(N_PAD, 256), ob.reshape(N_PAD, 256)],
                           axis=1)


def _dinv_body(deg_ref, out_ref):
    d = deg_ref[:, 0:1] + deg_ref[:, 1:2] + 1.0
    out_ref[...] = jax.lax.rsqrt(d)


def _dinv(deg2t):
    return pl.pallas_call(
        _dinv_body,
        in_specs=[pl.BlockSpec((N_PAD, 2), lambda: (0, 0))],
        out_specs=pl.BlockSpec((N_PAD, 1), lambda: (0, 0)),
        out_shape=jax.ShapeDtypeStruct((N_PAD, 1), jnp.float32),
    )(deg2t)


def _head_body(x_ref, w_ref, b_ref, dinv_ref, h_ref, g_ref):
    h = jnp.maximum(
        jnp.dot(x_ref[...], w_ref[...], preferred_element_type=jnp.float32)
        + b_ref[...], 0.0)
    h_ref[...] = h
    g_ref[...] = h * dinv_ref[...]


def _head(x, W, b, dinv, blk=1280):
    n = x.shape[0]
    return pl.pallas_call(
        _head_body,
        grid=(n // blk,),
        in_specs=[
            pl.BlockSpec((blk, x.shape[1]), lambda i: (i, 0)),
            pl.BlockSpec(W.shape, lambda i: (0, 0)),
            pl.BlockSpec((1, b.shape[1]), lambda i: (0, 0)),
            pl.BlockSpec((blk, 1), lambda i: (i, 0)),
        ],
        out_specs=[
            pl.BlockSpec((blk, W.shape[1]), lambda i: (i, 0)),
            pl.BlockSpec((blk, W.shape[1]), lambda i: (i, 0)),
        ],
        out_shape=[
            jax.ShapeDtypeStruct((n, W.shape[1]), jnp.float32),
            jax.ShapeDtypeStruct((n, W.shape[1]), jnp.float32),
        ],
    )(x, W, b, dinv)


def _layer0_body(agg_ref, g_ref, dinv_ref, w_ref, b_ref, h_ref, g2_ref):
    u = dinv_ref[...] * (agg_ref[0] + agg_ref[1] + g_ref[...])
    m = jnp.dot(u, w_ref[...], preferred_element_type=jnp.float32) + b_ref[...]
    h = jnp.maximum(m, 0.0)
    h_ref[...] = h
    g2_ref[...] = h * dinv_ref[...]


def _layer0(agg, g, dinv, W, b, blk=1280):
    n, d = g.shape
    h = W.shape[1]
    return pl.pallas_call(
        _layer0_body,
        grid=(n // blk,),
        in_specs=[
            pl.BlockSpec((2, blk, d), lambda i: (0, i, 0)),
            pl.BlockSpec((blk, d), lambda i: (i, 0)),
            pl.BlockSpec((blk, 1), lambda i: (i, 0)),
            pl.BlockSpec((d, h), lambda i: (0, 0)),
            pl.BlockSpec((1, h), lambda i: (0, 0)),
        ],
        out_specs=[
            pl.BlockSpec((blk, h), lambda i: (i, 0)),
            pl.BlockSpec((blk, h), lambda i: (i, 0)),
        ],
        out_shape=[
            jax.ShapeDtypeStruct((n, h), jnp.float32),
            jax.ShapeDtypeStruct((n, h), jnp.float32),
        ],
    )(agg, g, dinv, W, b)


def _layer_body(agg_ref, g_ref, dinv_ref, w_ref, b_ref, h_ref, g2_ref):
    u = dinv_ref[...] * (agg_ref[...] + g_ref[...])
    m = jnp.dot(u, w_ref[...], preferred_element_type=jnp.float32) + b_ref[...]
    h = jnp.maximum(m, 0.0)
    h_ref[...] = h
    g2_ref[...] = h * dinv_ref[...]


def _layer(agg, g, dinv, W, b, blk=1280):
    n, d = agg.shape
    h = W.shape[1]
    return pl.pallas_call(
        _layer_body,
        grid=(n // blk,),
        in_specs=[
            pl.BlockSpec((blk, d), lambda i: (i, 0)),
            pl.BlockSpec((blk, d), lambda i: (i, 0)),
            pl.BlockSpec((blk, 1), lambda i: (i, 0)),
            pl.BlockSpec((d, h), lambda i: (0, 0)),
            pl.BlockSpec((1, h), lambda i: (0, 0)),
        ],
        out_specs=[
            pl.BlockSpec((blk, h), lambda i: (i, 0)),
            pl.BlockSpec((blk, h), lambda i: (i, 0)),
        ],
        out_shape=[
            jax.ShapeDtypeStruct((n, h), jnp.float32),
            jax.ShapeDtypeStruct((n, h), jnp.float32),
        ],
    )(agg, g, dinv, W, b)


def _mlp_body(h1_ref, h2_ref, h3_ref, w1_ref, b1_ref, w2_ref, b2_ref, out_ref):
    outs = []
    for ref in (h1_ref, h2_ref, h3_ref):
        t = jnp.maximum(
            jnp.dot(ref[...], w1_ref[...], preferred_element_type=jnp.float32)
            + b1_ref[...], 0.0)
        o = jnp.dot(t, w2_ref[...], preferred_element_type=jnp.float32) + b2_ref[...]
        outs.append(o[:, None, :])
    out_ref[...] = jnp.concatenate(outs, axis=1)


def _mlp(h1, h2, h3, Wo1, bo1, Wo2, bo2, blk=400):
    n, hdim = h1.shape
    odim = Wo2.shape[1]
    return pl.pallas_call(
        _mlp_body,
        grid=(N_NODES // blk,),
        in_specs=[
            pl.BlockSpec((blk, hdim), lambda i: (i, 0)),
            pl.BlockSpec((blk, hdim), lambda i: (i, 0)),
            pl.BlockSpec((blk, hdim), lambda i: (i, 0)),
            pl.BlockSpec(Wo1.shape, lambda i: (0, 0)),
            pl.BlockSpec((1, bo1.shape[1]), lambda i: (0, 0)),
            pl.BlockSpec(Wo2.shape, lambda i: (0, 0)),
            pl.BlockSpec((1, bo2.shape[1]), lambda i: (0, 0)),
        ],
        out_specs=pl.BlockSpec((blk, 3, odim), lambda i: (i, 0, 0)),
        out_shape=jax.ShapeDtypeStruct((N_NODES, 3, odim), jnp.float32),
    )(h1, h2, h3, Wo1, bo1, Wo2, bo2)


def _pool_body(h_ref, batch_ref, out_ref, acc_ref, cnt_ref):
    i = pl.program_id(0)
    nprog = pl.num_programs(0)

    @pl.when(i == 0)
    def _():
        acc_ref[...] = jnp.zeros_like(acc_ref)
        cnt_ref[...] = jnp.zeros_like(cnt_ref)

    b = batch_ref[...]
    oh = (b == jax.lax.broadcasted_iota(jnp.int32, (1, NUM_SEG), 1)
          ).astype(jnp.float32)
    acc_ref[...] += jnp.dot(oh.T, h_ref[...], preferred_element_type=jnp.float32)
    cnt_ref[...] += jnp.sum(oh, axis=0)[:, None]

    @pl.when(i == nprog - 1)
    def _():
        out_ref[...] = acc_ref[...] / jnp.maximum(cnt_ref[...], 1.0)


def _pool(h, batch2d, blk=2000):
    hdim = h.shape[1]
    return pl.pallas_call(
        _pool_body,
        grid=(N_NODES // blk,),
        in_specs=[
            pl.BlockSpec((blk, hdim), lambda i: (i, 0)),
            pl.BlockSpec((blk, 1), lambda i: (i, 0)),
        ],
        out_specs=pl.BlockSpec((NUM_SEG, hdim), lambda i: (0, 0)),
        out_shape=jax.ShapeDtypeStruct((NUM_SEG, hdim), jnp.float32),
        scratch_shapes=[
            pltpu.VMEM((NUM_SEG, hdim), jnp.float32),
            pltpu.VMEM((NUM_SEG, 1), jnp.float32),
        ],
    )(h, batch2d)


def kernel(x, edge_index, batch, edge_attr, W_head, b_head, W0, b0, W1, b1,
           W2, b2, Wo1, bo1, Wo2, bo2):
    npad_e = E_PAD - E_TOT
    pad_src = jnp.arange(npad_e, dtype=jnp.int32) % N_NODES
    pad_dst = N_PAD + jnp.arange(npad_e, dtype=jnp.int32) % 256
    src2d = jnp.concatenate(
        [edge_index[0].astype(jnp.int32), pad_src]).reshape(ER, EB)
    dst2d = jnp.concatenate(
        [edge_index[1].astype(jnp.int32), pad_dst]).reshape(ER, EB)
    xp = jnp.concatenate(
        [x, jnp.zeros((N_PAD - N_NODES, x.shape[1]), jnp.float32)], axis=0)
    batchp = batch[:, None].astype(jnp.int32)

    deg2 = _sc_deg(dst2d)
    dinv = _dinv(deg2.T)

    h0, g0 = _head(xp, W_head, b_head[None, :], dinv)
    agg0 = _sc_agg128(src2d, dst2d, g0)
    h1, g1 = _layer0(agg0, g0, dinv, W0, b0[None, :])
    agg1 = _sc_agg512(src2d, dst2d, g1)
    h2, g2 = _layer(agg1, g1, dinv, W1, b1[None, :])
    agg2 = _sc_agg512(src2d, dst2d, g2)
    h3, _ = _layer(agg2, g2, dinv, W2, b2[None, :])

    emb_n = _mlp(h1, h2, h3, Wo1, bo1[None, :], Wo2, bo2[None, :])
    emb_g = _pool(h3, batchp)

    return (emb_g[:, None, :], emb_n, None)


# speedup vs baseline: 7.5805x; 1.0860x over previous
import functools
import jax
import jax.numpy as jnp
from jax import lax
from jax.experimental import pallas as pl
from jax.experimental.pallas import tpu as pltpu
from jax.experimental.pallas import tpu_sc as plsc

N_NODES = 10000
N_PAD = 10240
NUM_SEG = 64
E_TOT = 320000
EB = 80
ER = 4096
E_PAD = ER * EB
N_SINK = 10496
PART = 2560
HPART = 5120
QTHRESH = 3968
QCAP = 4160

_MESH = plsc.VectorSubcoreMesh(core_axis_name="c", subcore_axis_name="s")
_SC_PARAMS = pltpu.CompilerParams(needs_layout_passes=False)
_NC = 2
_NS = 16


def _zero_vmem(ref, rows, cols):
    z = jnp.zeros((16,), jnp.float32)

    def body(i, _):
        r = i // (cols // 16)
        cidx = (i % (cols // 16)) * 16
        ref[r, pl.ds(cidx, 16)] = z
        return 0

    lax.fori_loop(0, rows * (cols // 16), body, 0)


def _zero_vmem3(ref):
    z = jnp.zeros((16,), jnp.float32)

    def body(i, _):
        r = i // 32
        q = (i % 32) // 8
        cidx = (i % 8) * 16
        ref[r, q, pl.ds(cidx, 16)] = z
        return 0

    lax.fori_loop(0, 16 * 4 * 8, body, 0)


def _sc_deg_body(dst_hbm, out_hbm, dst_v, ones_v, zb_v, deg_sh, deg_out_v):
    c = lax.axis_index("c")
    s = lax.axis_index("s")
    wid = s * _NC + c

    def zb(i, _):
        zb_v[pl.ds(i * 16, 16)] = jnp.zeros((16,), jnp.float32)
        return 0
    lax.fori_loop(0, 41, zb, 0)

    def ob(i, _):
        ones_v[pl.ds(i * 16, 16)] = jnp.ones((16,), jnp.float32)
        return 0
    lax.fori_loop(0, EB // 16, ob, 0)

    pltpu.sync_copy(zb_v, deg_sh.at[pl.ds(s * 656, 656)])
    plsc.subcore_barrier()

    rows = ER // (_NC * _NS)
    pltpu.sync_copy(dst_hbm.at[pl.ds(wid * rows, rows)], dst_v)

    def body(j, _):
        pltpu.sync_copy(ones_v, deg_sh.at[dst_v.at[j]], add=True)
        return 0
    lax.fori_loop(0, rows, body, 0)

    plsc.subcore_barrier()
    pltpu.sync_copy(deg_sh.at[pl.ds(s * 640, 640)], deg_out_v)
    pltpu.sync_copy(deg_out_v, out_hbm.at[c, pl.ds(s * 640, 640)])


def _sc_deg(dst2d):
    f = functools.partial(
        pl.kernel,
        out_type=jax.ShapeDtypeStruct((2, N_PAD), jnp.float32),
        compiler_params=_SC_PARAMS,
        mesh=_MESH,
        scratch_types=[
            pltpu.VMEM((ER // 32, EB), jnp.int32),
            pltpu.VMEM((EB,), jnp.float32),
            pltpu.VMEM((656,), jnp.float32),
            pltpu.VMEM_SHARED((N_SINK,), jnp.float32),
            pltpu.VMEM((640,), jnp.float32),
        ],
    )(_sc_deg_body)
    return f(dst2d)


def _sc_agg128_body(src_hbm, dst_hbm, g_hbm, out_hbm,
                    src_v, dst_v, bufs, agg_sh, out_v, gsems, ssems):
    c = lax.axis_index("c")
    s = lax.axis_index("s")
    wid = s * _NC + c

    _zero_vmem(out_v, 16, 128)
    for k in range(41):
        pltpu.sync_copy(out_v, agg_sh.at[pl.ds(s * 656 + k * 16, 16), :])
    plsc.subcore_barrier()

    rows = ER // (_NC * _NS)
    pltpu.sync_copy(src_hbm.at[pl.ds(wid * rows, rows)], src_v)
    pltpu.sync_copy(dst_hbm.at[pl.ds(wid * rows, rows)], dst_v)

    def body(j, _):
        gh = []
        for k in range(5):
            si = src_v[j, pl.ds(k * 16, 16)]
            gh.append(pltpu.async_copy(g_hbm.at[si], bufs[k], gsems[k]))
        sh = []
        for k in range(5):
            di = dst_v[j, pl.ds(k * 16, 16)]
            gh[k].wait()
            sh.append(pltpu.async_copy(bufs[k], agg_sh.at[di], ssems[k],
                                       add=True))
        for k in range(5):
            sh[k].wait()
        return 0
    lax.fori_loop(0, rows, body, 0)

    plsc.subcore_barrier()

    def obody(k, _):
        pltpu.sync_copy(agg_sh.at[pl.ds(s * 640 + k * 16, 16), :], out_v)
        pltpu.sync_copy(out_v, out_hbm.at[c, pl.ds(s * 640 + k * 16, 16), :])
        return 0
    lax.fori_loop(0, 40, obody, 0)


def _sc_agg128(src2d, dst2d, g):
    f = functools.partial(
        pl.kernel,
        out_type=jax.ShapeDtypeStruct((2, N_PAD, 128), jnp.float32),
        compiler_params=_SC_PARAMS,
        mesh=_MESH,
        scratch_types=[
            pltpu.VMEM((ER // 32, EB), jnp.int32),
            pltpu.VMEM((ER // 32, EB), jnp.int32),
            [pltpu.VMEM((16, 128), jnp.float32) for _ in range(5)],
            pltpu.VMEM_SHARED((N_SINK, 128), jnp.float32),
            pltpu.VMEM((16, 128), jnp.float32),
            [pltpu.SemaphoreType.DMA for _ in range(5)],
            [pltpu.SemaphoreType.DMA for _ in range(5)],
        ],
    )(_sc_agg128_body)
    return f(src2d, dst2d, g)


def _zero_vmem3h(ref):
    z = jnp.zeros((16,), jnp.float32)

    def body(i, _):
        r = i // 16
        q = (i % 16) // 8
        cidx = (i % 8) * 16
        ref[r, q, pl.ds(cidx, 16)] = z
        return 0

    lax.fori_loop(0, 16 * 2 * 8, body, 0)


def _sc_agg512_body(src_hbm, dst_hbm, ga_hbm, gb_hbm, outa_hbm, outb_hbm,
                    sbuf, dbuf, q_v, sidx_v, didx_v, bufs, part_sh,
                    gsems, ssems):
    c = lax.axis_index("c")
    s = lax.axis_index("s")

    rows = ER // _NS
    _zero_vmem3h(bufs[2])
    mask14 = jnp.full((16,), 16383, jnp.int32)
    halfu = jnp.full((16,), HPART, jnp.uint32)
    lo_v = jnp.full((16,), c * HPART, jnp.int32)

    for h, (g_hbm, out_hbm) in enumerate(((ga_hbm, outa_hbm),
                                          (gb_hbm, outb_hbm))):
        def zbody(k, _):
            pltpu.sync_copy(bufs[2],
                            part_sh.at[pl.ds(s * 320 + k * 16, 16)])
            return 0
        lax.fori_loop(0, 20, zbody, 0)
        plsc.subcore_barrier()

        def fchunk(t, _):
            pltpu.sync_copy(src_hbm.at[pl.ds(s * rows + t * 16, 16)], sbuf)
            pltpu.sync_copy(dst_hbm.at[pl.ds(s * rows + t * 16, 16)], dbuf)
            sinkrel = jnp.full((16,), HPART + s, jnp.int32)
            for r in range(16):
                for k in range(5):
                    dv = dbuf[r, pl.ds(k * 16, 16)]
                    sv = sbuf[r, pl.ds(k * 16, 16)]
                    rel = dv - lo_v
                    m = plsc.bitcast(rel, jnp.uint32) < halfu
                    relx = jnp.where(m, rel, sinkrel)
                    packed = jnp.bitwise_or(jnp.left_shift(relx, 14), sv)
                    q_v[pl.ds((r * 5 + k) * 16, 16)] = packed

            def gb2(t2, _):
                base = t2 * 128
                for k in range(2):
                    for j in range(4):
                        qv = q_v[pl.ds(base + k * 64 + j * 16, 16)]
                        sidx_v[k, pl.ds(j * 16, 16)] = jnp.bitwise_and(
                            qv, mask14)
                        didx_v[k, pl.ds(j * 16, 16)] = jnp.right_shift(qv, 14)
                gh = []
                for k in range(2):
                    gh.append(pltpu.async_copy(g_hbm.at[sidx_v.at[k]],
                                               bufs[k], gsems[k]))
                sh = []
                for k in range(2):
                    gh[k].wait()
                    sh.append(pltpu.async_copy(bufs[k],
                                               part_sh.at[didx_v.at[k]],
                                               ssems[k], add=True))
                for k in range(2):
                    sh[k].wait()
                return 0
            lax.fori_loop(0, 10, gb2, 0)
            return 0
        lax.fori_loop(0, rows // 16, fchunk, 0)

        plsc.subcore_barrier()
        def obody(k, _):
            r = s * 320 + k * 16
            pltpu.sync_copy(part_sh.at[pl.ds(r, 16)], bufs[3])
            pltpu.sync_copy(bufs[3], out_hbm.at[pl.ds(c * HPART + r, 16)])
            return 0
        lax.fori_loop(0, 20, obody, 0)
        _zero_vmem3h(bufs[2])
        plsc.subcore_barrier()


def _sc_agg512(src2d, dst2d, g):
    ga = g[:, :256].reshape(N_PAD, 2, 128)
    gb = g[:, 256:].reshape(N_PAD, 2, 128)
    f = functools.partial(
        pl.kernel,
        out_type=[jax.ShapeDtypeStruct((N_PAD, 2, 128), jnp.float32),
                  jax.ShapeDtypeStruct((N_PAD, 2, 128), jnp.float32)],
        compiler_params=_SC_PARAMS,
        mesh=_MESH,
        scratch_types=[
            pltpu.VMEM((16, EB), jnp.int32),
            pltpu.VMEM((16, EB), jnp.int32),
            pltpu.VMEM((1280,), jnp.int32),
            pltpu.VMEM((2, 64), jnp.int32),
            pltpu.VMEM((2, 64), jnp.int32),
            [pltpu.VMEM((64, 2, 128), jnp.float32) for _ in range(2)] +
            [pltpu.VMEM((16, 2, 128), jnp.float32) for _ in range(2)],
            pltpu.VMEM_SHARED((HPART + 16, 2, 128), jnp.float32),
            [pltpu.SemaphoreType.DMA for _ in range(2)],
            [pltpu.SemaphoreType.DMA for _ in range(2)],
        ],
    )(_sc_agg512_body)
    oa, ob = f(src2d, dst2d, ga, gb)
    return jnp.concatenate([oa.reshape(N_PAD, 256), ob.reshape(N_PAD, 256)],
                           axis=1)


def _dinv_body(deg_ref, out_ref):
    d = deg_ref[:, 0:1] + deg_ref[:, 1:2] + 1.0
    out_ref[...] = jax.lax.rsqrt(d)


def _dinv(deg2t):
    return pl.pallas_call(
        _dinv_body,
        in_specs=[pl.BlockSpec((N_PAD, 2), lambda: (0, 0))],
        out_specs=pl.BlockSpec((N_PAD, 1), lambda: (0, 0)),
        out_shape=jax.ShapeDtypeStruct((N_PAD, 1), jnp.float32),
    )(deg2t)


def _head_body(x_ref, w_ref, b_ref, dinv_ref, h_ref, g_ref):
    h = jnp.maximum(
        jnp.dot(x_ref[...], w_ref[...], preferred_element_type=jnp.float32)
        + b_ref[...], 0.0)
    h_ref[...] = h
    g_ref[...] = h * dinv_ref[...]


def _head(x, W, b, dinv, blk=1280):
    n = x.shape[0]
    return pl.pallas_call(
        _head_body,
        grid=(n // blk,),
        in_specs=[
            pl.BlockSpec((blk, x.shape[1]), lambda i: (i, 0)),
            pl.BlockSpec(W.shape, lambda i: (0, 0)),
            pl.BlockSpec((1, b.shape[1]), lambda i: (0, 0)),
            pl.BlockSpec((blk, 1), lambda i: (i, 0)),
        ],
        out_specs=[
            pl.BlockSpec((blk, W.shape[1]), lambda i: (i, 0)),
            pl.BlockSpec((blk, W.shape[1]), lambda i: (i, 0)),
        ],
        out_shape=[
            jax.ShapeDtypeStruct((n, W.shape[1]), jnp.float32),
            jax.ShapeDtypeStruct((n, W.shape[1]), jnp.float32),
        ],
    )(x, W, b, dinv)


def _layer0_body(agg_ref, g_ref, dinv_ref, w_ref, b_ref, h_ref, g2_ref):
    u = dinv_ref[...] * (agg_ref[0] + agg_ref[1] + g_ref[...])
    m = jnp.dot(u, w_ref[...], preferred_element_type=jnp.float32) + b_ref[...]
    h = jnp.maximum(m, 0.0)
    h_ref[...] = h
    g2_ref[...] = h * dinv_ref[...]


def _layer0(agg, g, dinv, W, b, blk=1280):
    n, d = g.shape
    h = W.shape[1]
    return pl.pallas_call(
        _layer0_body,
        grid=(n // blk,),
        in_specs=[
            pl.BlockSpec((2, blk, d), lambda i: (0, i, 0)),
            pl.BlockSpec((blk, d), lambda i: (i, 0)),
            pl.BlockSpec((blk, 1), lambda i: (i, 0)),
            pl.BlockSpec((d, h), lambda i: (0, 0)),
            pl.BlockSpec((1, h), lambda i: (0, 0)),
        ],
        out_specs=[
            pl.BlockSpec((blk, h), lambda i: (i, 0)),
            pl.BlockSpec((blk, h), lambda i: (i, 0)),
        ],
        out_shape=[
            jax.ShapeDtypeStruct((n, h), jnp.float32),
            jax.ShapeDtypeStruct((n, h), jnp.float32),
        ],
    )(agg, g, dinv, W, b)


def _layer_body(agg_ref, g_ref, dinv_ref, w_ref, b_ref, h_ref, g2_ref):
    u = dinv_ref[...] * (agg_ref[...] + g_ref[...])
    m = jnp.dot(u, w_ref[...], preferred_element_type=jnp.float32) + b_ref[...]
    h = jnp.maximum(m, 0.0)
    h_ref[...] = h
    g2_ref[...] = h * dinv_ref[...]


def _layer(agg, g, dinv, W, b, blk=1280):
    n, d = agg.shape
    h = W.shape[1]
    return pl.pallas_call(
        _layer_body,
        grid=(n // blk,),
        in_specs=[
            pl.BlockSpec((blk, d), lambda i: (i, 0)),
            pl.BlockSpec((blk, d), lambda i: (i, 0)),
            pl.BlockSpec((blk, 1), lambda i: (i, 0)),
            pl.BlockSpec((d, h), lambda i: (0, 0)),
            pl.BlockSpec((1, h), lambda i: (0, 0)),
        ],
        out_specs=[
            pl.BlockSpec((blk, h), lambda i: (i, 0)),
            pl.BlockSpec((blk, h), lambda i: (i, 0)),
        ],
        out_shape=[
            jax.ShapeDtypeStruct((n, h), jnp.float32),
            jax.ShapeDtypeStruct((n, h), jnp.float32),
        ],
    )(agg, g, dinv, W, b)


def _mlp_body(h1_ref, h2_ref, h3_ref, w1_ref, b1_ref, w2_ref, b2_ref, out_ref):
    outs = []
    for ref in (h1_ref, h2_ref, h3_ref):
        t = jnp.maximum(
            jnp.dot(ref[...], w1_ref[...], preferred_element_type=jnp.float32)
            + b1_ref[...], 0.0)
        o = jnp.dot(t, w2_ref[...], preferred_element_type=jnp.float32) + b2_ref[...]
        outs.append(o[:, None, :])
    out_ref[...] = jnp.concatenate(outs, axis=1)


def _mlp(h1, h2, h3, Wo1, bo1, Wo2, bo2, blk=400):
    n, hdim = h1.shape
    odim = Wo2.shape[1]
    return pl.pallas_call(
        _mlp_body,
        grid=(N_NODES // blk,),
        in_specs=[
            pl.BlockSpec((blk, hdim), lambda i: (i, 0)),
            pl.BlockSpec((blk, hdim), lambda i: (i, 0)),
            pl.BlockSpec((blk, hdim), lambda i: (i, 0)),
            pl.BlockSpec(Wo1.shape, lambda i: (0, 0)),
            pl.BlockSpec((1, bo1.shape[1]), lambda i: (0, 0)),
            pl.BlockSpec(Wo2.shape, lambda i: (0, 0)),
            pl.BlockSpec((1, bo2.shape[1]), lambda i: (0, 0)),
        ],
        out_specs=pl.BlockSpec((blk, 3, odim), lambda i: (i, 0, 0)),
        out_shape=jax.ShapeDtypeStruct((N_NODES, 3, odim), jnp.float32),
    )(h1, h2, h3, Wo1, bo1, Wo2, bo2)


def _pool_body(h_ref, batch_ref, out_ref, acc_ref, cnt_ref):
    i = pl.program_id(0)
    nprog = pl.num_programs(0)

    @pl.when(i == 0)
    def _():
        acc_ref[...] = jnp.zeros_like(acc_ref)
        cnt_ref[...] = jnp.zeros_like(cnt_ref)

    b = batch_ref[...]
    oh = (b == jax.lax.broadcasted_iota(jnp.int32, (1, NUM_SEG), 1)
          ).astype(jnp.float32)
    acc_ref[...] += jnp.dot(oh.T, h_ref[...], preferred_element_type=jnp.float32)
    cnt_ref[...] += jnp.sum(oh, axis=0)[:, None]

    @pl.when(i == nprog - 1)
    def _():
        out_ref[...] = acc_ref[...] / jnp.maximum(cnt_ref[...], 1.0)


def _pool(h, batch2d, blk=2000):
    hdim = h.shape[1]
    return pl.pallas_call(
        _pool_body,
        grid=(N_NODES // blk,),
        in_specs=[
            pl.BlockSpec((blk, hdim), lambda i: (i, 0)),
            pl.BlockSpec((blk, 1), lambda i: (i, 0)),
        ],
        out_specs=pl.BlockSpec((NUM_SEG, hdim), lambda i: (0, 0)),
        out_shape=jax.ShapeDtypeStruct((NUM_SEG, hdim), jnp.float32),
        scratch_shapes=[
            pltpu.VMEM((NUM_SEG, hdim), jnp.float32),
            pltpu.VMEM((NUM_SEG, 1), jnp.float32),
        ],
    )(h, batch2d)


def kernel(x, edge_index, batch, edge_attr, W_head, b_head, W0, b0, W1, b1,
           W2, b2, Wo1, bo1, Wo2, bo2):
    npad_e = E_PAD - E_TOT
    pad_src = jnp.arange(npad_e, dtype=jnp.int32) % N_NODES
    pad_dst = N_PAD + jnp.arange(npad_e, dtype=jnp.int32) % 256
    src2d = jnp.concatenate(
        [edge_index[0].astype(jnp.int32), pad_src]).reshape(ER, EB)
    dst2d = jnp.concatenate(
        [edge_index[1].astype(jnp.int32), pad_dst]).reshape(ER, EB)
    xp = jnp.concatenate(
        [x, jnp.zeros((N_PAD - N_NODES, x.shape[1]), jnp.float32)], axis=0)
    batchp = batch[:, None].astype(jnp.int32)

    deg2 = _sc_deg(dst2d)
    dinv = _dinv(deg2.T)

    h0, g0 = _head(xp, W_head, b_head[None, :], dinv)
    agg0 = _sc_agg128(src2d, dst2d, g0)
    h1, g1 = _layer0(agg0, g0, dinv, W0, b0[None, :])
    agg1 = _sc_agg512(src2d, dst2d, g1)
    h2, g2 = _layer(agg1, g1, dinv, W1, b1[None, :])
    agg2 = _sc_agg512(src2d, dst2d, g2)
    h3, _ = _layer(agg2, g2, dinv, W2, b2[None, :])

    emb_n = _mlp(h1, h2, h3, Wo1, bo1[None, :], Wo2, bo2[None, :])
    emb_g = _pool(h3, batchp)

    return (emb_g[:, None, :], emb_n, None)
